# Initial kernel scaffold; baseline (speedup 1.0000x reference)
import jax
import jax.numpy as jnp
from jax.experimental import pallas as pl


def kernel(x, x_mark, W_conv, pe, hour_t, weekday_t, day_t, month_t):
    raise NotImplementedError("write your pallas kernel here")


# speedup vs baseline: 10.0770x; 10.0770x over previous
import jax
import jax.numpy as jnp
from jax import lax
from jax.experimental import pallas as pl

B, L, C_IN, D_MODEL = 16, 4096, 7, 1024
TL = 512

_OFF_MONTH, _OFF_DAY, _OFF_WEEKDAY, _OFF_HOUR = 0, 13, 45, 52
_T_ROWS = 76


def _embed_block(xw_ref, idx_ref, pe_ref, wc_ref, tc_ref, out_ref):
    xwin = xw_ref[0]
    conv = jnp.dot(xwin, wc_ref[...], preferred_element_type=jnp.float32)

    idx = idx_ref[0]
    iota = lax.broadcasted_iota(jnp.int32, (TL, _T_ROWS), 1)
    oh = ((iota == idx[:, 0:1] + _OFF_MONTH)
          | (iota == idx[:, 1:2] + _OFF_DAY)
          | (iota == idx[:, 2:3] + _OFF_WEEKDAY)
          | (iota == idx[:, 3:4] + _OFF_HOUR)).astype(jnp.float32)
    temporal = jnp.dot(oh, tc_ref[...], preferred_element_type=jnp.float32)

    out_ref[0] = conv + temporal + pe_ref[...]


def kernel(x, x_mark, W_conv, pe, hour_t, weekday_t, day_t, month_t):
    xw = jnp.concatenate(
        [jnp.roll(x, 1, axis=1), x, jnp.roll(x, -1, axis=1)], axis=-1
    )
    wc = jnp.transpose(W_conv, (2, 1, 0)).reshape(3 * C_IN, D_MODEL)
    tcat = jnp.concatenate([month_t, day_t, weekday_t, hour_t], axis=0)
    pe4 = pe[:L]

    nl = L // TL
    grid = (nl, B)
    out = pl.pallas_call(
        _embed_block,
        grid=grid,
        in_specs=[
            pl.BlockSpec((1, TL, 3 * C_IN), lambda l, b: (b, l, 0)),
            pl.BlockSpec((1, TL, 4), lambda l, b: (b, l, 0)),
            pl.BlockSpec((TL, D_MODEL), lambda l, b: (l, 0)),
            pl.BlockSpec((3 * C_IN, D_MODEL), lambda l, b: (0, 0)),
            pl.BlockSpec((_T_ROWS, D_MODEL), lambda l, b: (0, 0)),
        ],
        out_specs=pl.BlockSpec((1, TL, D_MODEL), lambda l, b: (b, l, 0)),
        out_shape=jax.ShapeDtypeStruct((B, L, D_MODEL), jnp.float32),
    )(xw, x_mark, pe4, wc, tcat)
    return out


# speedup vs baseline: 11.9997x; 1.1908x over previous
import jax
import jax.numpy as jnp
from jax import lax
from jax.experimental import pallas as pl

B, L, C_IN, D_MODEL = 16, 4096, 7, 1024
TL = 1024

_OFF_MONTH, _OFF_DAY, _OFF_WEEKDAY, _OFF_HOUR = 0, 13, 45, 52
_T_ROWS = 76


def _embed_block(xw_ref, idx_ref, pe_ref, wc_ref, tc_ref, out_ref):
    xwin = xw_ref[0]
    conv = jnp.dot(xwin, wc_ref[...], preferred_element_type=jnp.float32)

    idx = idx_ref[0]
    iota = lax.broadcasted_iota(jnp.int32, (TL, _T_ROWS), 1)
    oh = ((iota == idx[:, 0:1] + _OFF_MONTH)
          | (iota == idx[:, 1:2] + _OFF_DAY)
          | (iota == idx[:, 2:3] + _OFF_WEEKDAY)
          | (iota == idx[:, 3:4] + _OFF_HOUR)).astype(jnp.float32)
    temporal = jnp.dot(oh, tc_ref[...], preferred_element_type=jnp.float32)

    out_ref[0] = conv + temporal + pe_ref[...]


def kernel(x, x_mark, W_conv, pe, hour_t, weekday_t, day_t, month_t):
    xw = jnp.concatenate(
        [jnp.roll(x, 1, axis=1), x, jnp.roll(x, -1, axis=1)], axis=-1
    )
    wc = jnp.transpose(W_conv, (2, 1, 0)).reshape(3 * C_IN, D_MODEL)
    tcat = jnp.concatenate([month_t, day_t, weekday_t, hour_t], axis=0)
    pe4 = pe[:L]

    nl = L // TL
    grid = (nl, B)
    out = pl.pallas_call(
        _embed_block,
        grid=grid,
        in_specs=[
            pl.BlockSpec((1, TL, 3 * C_IN), lambda l, b: (b, l, 0)),
            pl.BlockSpec((1, TL, 4), lambda l, b: (b, l, 0)),
            pl.BlockSpec((TL, D_MODEL), lambda l, b: (l, 0)),
            pl.BlockSpec((3 * C_IN, D_MODEL), lambda l, b: (0, 0)),
            pl.BlockSpec((_T_ROWS, D_MODEL), lambda l, b: (0, 0)),
        ],
        out_specs=pl.BlockSpec((1, TL, D_MODEL), lambda l, b: (b, l, 0)),
        out_shape=jax.ShapeDtypeStruct((B, L, D_MODEL), jnp.float32),
    )(xw, x_mark, pe4, wc, tcat)
    return out


# speedup vs baseline: 13.3100x; 1.1092x over previous
import jax
import jax.numpy as jnp
from jax import lax
from jax.experimental import pallas as pl

B, L, C_IN, D_MODEL = 16, 4096, 7, 1024
TL = 2048

_OFF_MONTH, _OFF_DAY, _OFF_WEEKDAY, _OFF_HOUR = 0, 13, 45, 52
_T_ROWS = 76


def _embed_block(xw_ref, idx_ref, pe_ref, wc_ref, tc_ref, out_ref):
    xwin = xw_ref[0]
    conv = jnp.dot(xwin, wc_ref[...], preferred_element_type=jnp.float32)

    idx = idx_ref[0]
    iota = lax.broadcasted_iota(jnp.int32, (TL, _T_ROWS), 1)
    oh = ((iota == idx[:, 0:1] + _OFF_MONTH)
          | (iota == idx[:, 1:2] + _OFF_DAY)
          | (iota == idx[:, 2:3] + _OFF_WEEKDAY)
          | (iota == idx[:, 3:4] + _OFF_HOUR)).astype(jnp.float32)
    temporal = jnp.dot(oh, tc_ref[...], preferred_element_type=jnp.float32)

    out_ref[0] = conv + temporal + pe_ref[...]


def kernel(x, x_mark, W_conv, pe, hour_t, weekday_t, day_t, month_t):
    xw = jnp.concatenate(
        [jnp.roll(x, 1, axis=1), x, jnp.roll(x, -1, axis=1)], axis=-1
    )
    wc = jnp.transpose(W_conv, (2, 1, 0)).reshape(3 * C_IN, D_MODEL)
    tcat = jnp.concatenate([month_t, day_t, weekday_t, hour_t], axis=0)
    pe4 = pe[:L]

    nl = L // TL
    grid = (nl, B)
    out = pl.pallas_call(
        _embed_block,
        grid=grid,
        in_specs=[
            pl.BlockSpec((1, TL, 3 * C_IN), lambda l, b: (b, l, 0)),
            pl.BlockSpec((1, TL, 4), lambda l, b: (b, l, 0)),
            pl.BlockSpec((TL, D_MODEL), lambda l, b: (l, 0)),
            pl.BlockSpec((3 * C_IN, D_MODEL), lambda l, b: (0, 0)),
            pl.BlockSpec((_T_ROWS, D_MODEL), lambda l, b: (0, 0)),
        ],
        out_specs=pl.BlockSpec((1, TL, D_MODEL), lambda l, b: (b, l, 0)),
        out_shape=jax.ShapeDtypeStruct((B, L, D_MODEL), jnp.float32),
    )(xw, x_mark, pe4, wc, tcat)
    return out


# speedup vs baseline: 13.4703x; 1.0120x over previous
import jax
import jax.numpy as jnp
from jax import lax
from jax.experimental import pallas as pl

B, L, C_IN, D_MODEL = 16, 4096, 7, 1024
TL = 2048

_OFF_MONTH, _OFF_DAY, _OFF_WEEKDAY, _OFF_HOUR = 0, 13, 45, 52
_T_ROWS = 76


def _embed_block(x_ref, idx_ref, pe_ref, wc_ref, tc_ref, out_ref):
    l = pl.program_id(0)
    start = l * TL
    main = x_ref[0, pl.ds(start, TL)]
    row_prev = x_ref[0, pl.ds((start - 1) % L, 1)]
    row_next = x_ref[0, pl.ds((start + TL) % L, 1)]
    shift_m1 = jnp.concatenate([row_prev, main[:-1]], axis=0)
    shift_p1 = jnp.concatenate([main[1:], row_next], axis=0)
    xwin = jnp.concatenate([shift_m1, main, shift_p1], axis=1)
    conv = jnp.dot(xwin, wc_ref[...], preferred_element_type=jnp.float32)

    idx = idx_ref[0]
    iota = lax.broadcasted_iota(jnp.int32, (TL, _T_ROWS), 1)
    oh = ((iota == idx[:, 0:1] + _OFF_MONTH)
          | (iota == idx[:, 1:2] + _OFF_DAY)
          | (iota == idx[:, 2:3] + _OFF_WEEKDAY)
          | (iota == idx[:, 3:4] + _OFF_HOUR)).astype(jnp.float32)
    temporal = jnp.dot(oh, tc_ref[...], preferred_element_type=jnp.float32)

    out_ref[0] = conv + temporal + pe_ref[...]


def kernel(x, x_mark, W_conv, pe, hour_t, weekday_t, day_t, month_t):
    wc = jnp.transpose(W_conv, (2, 1, 0)).reshape(3 * C_IN, D_MODEL)
    tcat = jnp.concatenate([month_t, day_t, weekday_t, hour_t], axis=0)

    nl = L // TL
    grid = (nl, B)
    out = pl.pallas_call(
        _embed_block,
        grid=grid,
        in_specs=[
            pl.BlockSpec((1, L, C_IN), lambda l, b: (b, 0, 0)),
            pl.BlockSpec((1, TL, 4), lambda l, b: (b, l, 0)),
            pl.BlockSpec((TL, D_MODEL), lambda l, b: (l, 0)),
            pl.BlockSpec((3 * C_IN, D_MODEL), lambda l, b: (0, 0)),
            pl.BlockSpec((_T_ROWS, D_MODEL), lambda l, b: (0, 0)),
        ],
        out_specs=pl.BlockSpec((1, TL, D_MODEL), lambda l, b: (b, l, 0)),
        out_shape=jax.ShapeDtypeStruct((B, L, D_MODEL), jnp.float32),
    )(x, x_mark, pe, wc, tcat)
    return out
